# Initial kernel scaffold; baseline (speedup 1.0000x reference)
import functools

import jax
import jax.numpy as jnp
from jax import lax
from jax.experimental import pallas as pl
from jax.experimental.pallas import tpu as pltpu
from jax.experimental.pallas import tpu_sc as plsc

F32 = jnp.float32

_NC = 2
_NS = 16
_LANES = 16


def _deg_call(dst, ones, zeros16, n, e):
    chunk = 40
    per_tile = e // (_NC * _NS)
    nch = per_tile // chunk
    rows_t = n // _NS

    def body(dst_hbm, ones_hbm, zeros_hbm, out_hbm, idx_d, ones_v, acc_sh):
        c = lax.axis_index("c")
        s = lax.axis_index("s")
        pltpu.sync_copy(zeros_hbm.at[pl.ds(s * rows_t, rows_t)],
                        acc_sh.at[pl.ds(s * rows_t, rows_t)])
        pltpu.sync_copy(ones_hbm, ones_v)
        plsc.subcore_barrier()
        base = (c * _NS + s) * per_tile

        def step(i, carry):
            e0 = base + i * chunk
            pltpu.sync_copy(dst_hbm.at[pl.ds(e0, chunk)], idx_d)
            pltpu.sync_copy(ones_v, acc_sh.at[idx_d], add=True)
            return carry

        lax.fori_loop(0, nch, step, 0)
        plsc.subcore_barrier()
        pltpu.sync_copy(acc_sh.at[pl.ds(s * rows_t, rows_t)],
                        out_hbm.at[pl.ds(c * n + s * rows_t, rows_t)])

    kfn = pl.kernel(
        body,
        out_type=jax.ShapeDtypeStruct((_NC * n, _LANES), F32),
        mesh=plsc.VectorSubcoreMesh(core_axis_name="c", subcore_axis_name="s"),
        scratch_types=[
            pltpu.VMEM((chunk,), jnp.int32),
            pltpu.VMEM((chunk, _LANES), F32),
            pltpu.VMEM_SHARED((n, _LANES), F32),
        ],
    )
    return kfn(dst, ones, zeros16)


def _edge_call(u2, src, dst, zeros128, n, e):
    chunk = 80
    per_tile = e // _NS
    nch = per_tile // chunk
    rows_t = n // _NS
    half = u2.shape[1]

    def body(u_hbm, src_hbm, dst_hbm, zeros_hbm, out_hbm,
             idx_s, idx_d, rows_v, acc_sh, sem):
        c = lax.axis_index("c")
        s = lax.axis_index("s")
        pltpu.sync_copy(zeros_hbm.at[pl.ds(s * rows_t, rows_t)],
                        acc_sh.at[pl.ds(s * rows_t, rows_t)])
        plsc.subcore_barrier()
        base = s * per_tile
        row_off = c * n

        def step(i, carry):
            e0 = base + i * chunk
            pltpu.sync_copy(src_hbm.at[pl.ds(e0, chunk)], idx_s)
            pltpu.sync_copy(dst_hbm.at[pl.ds(e0, chunk)], idx_d)
            for j in range(chunk // _LANES):
                v = idx_s[pl.ds(j * _LANES, _LANES)]
                idx_s[pl.ds(j * _LANES, _LANES)] = v + row_off
            pltpu.async_copy(u_hbm.at[idx_s], rows_v, sem).wait()
            pltpu.sync_copy(rows_v, acc_sh.at[idx_d], add=True)
            return carry

        lax.fori_loop(0, nch, step, 0)
        plsc.subcore_barrier()
        pltpu.sync_copy(acc_sh.at[pl.ds(s * rows_t, rows_t)],
                        out_hbm.at[pl.ds(row_off + s * rows_t, rows_t)])

    kfn = pl.kernel(
        body,
        out_type=jax.ShapeDtypeStruct((_NC * n, half), F32),
        mesh=plsc.VectorSubcoreMesh(core_axis_name="c", subcore_axis_name="s"),
        scratch_types=[
            pltpu.VMEM((chunk,), jnp.int32),
            pltpu.VMEM((chunk,), jnp.int32),
            pltpu.VMEM((chunk, half), F32),
            pltpu.VMEM_SHARED((n, half), F32),
            pltpu.SemaphoreType.DMA,
        ],
    )
    return kfn(u2, src, dst, zeros128)


def _deg_terms(cnt_blk):
    deg = 1.0 + cnt_blk[0, :, 0:1] + cnt_blk[1, :, 0:1]
    return lax.rsqrt(deg), 1.0 / deg


def _k1_body(x_ref, w_ref, cnt_ref, h_ref, u_ref):
    h = jnp.dot(x_ref[...], w_ref[...], preferred_element_type=F32)
    invs, _ = _deg_terms(cnt_ref[...])
    h_ref[...] = h
    u = invs * h
    hh = u.shape[1] // 2
    u_ref[0] = u[:, :hh]
    u_ref[1] = u[:, hh:]


def _k1(x, W1, cnt3, n, bm):
    grid = (n // bm,)
    d = x.shape[1]
    hdim = W1.shape[1]
    return pl.pallas_call(
        _k1_body,
        grid=grid,
        in_specs=[
            pl.BlockSpec((bm, d), lambda m: (m, 0)),
            pl.BlockSpec((d, hdim), lambda m: (0, 0)),
            pl.BlockSpec((_NC, bm, _LANES), lambda m: (0, m, 0)),
        ],
        out_specs=[
            pl.BlockSpec((bm, hdim), lambda m: (m, 0)),
            pl.BlockSpec((_NC, bm, hdim // 2), lambda m: (0, m, 0)),
        ],
        out_shape=[
            jax.ShapeDtypeStruct((n, hdim), F32),
            jax.ShapeDtypeStruct((_NC, n, hdim // 2), F32),
        ],
    )(x, W1, cnt3)


def _k2_body(h_ref, s_ref, cnt_ref, b_ref, w_ref, h2_ref, u_ref):
    sfull = jnp.concatenate([s_ref[0], s_ref[1]], axis=1)
    invs, invd = _deg_terms(cnt_ref[...])
    pre = invs * sfull + invd * h_ref[...] + b_ref[...]
    a = jnp.maximum(pre, 0.0)
    h2 = jnp.dot(a, w_ref[...], preferred_element_type=F32)
    h2_ref[...] = h2
    u = invs * h2
    hh = u.shape[1] // 2
    u_ref[0] = u[:, :hh]
    u_ref[1] = u[:, hh:]


def _k2(h1, s3, cnt3, b1, W2, n, bm):
    grid = (n // bm,)
    hdim = h1.shape[1]
    return pl.pallas_call(
        _k2_body,
        grid=grid,
        in_specs=[
            pl.BlockSpec((bm, hdim), lambda m: (m, 0)),
            pl.BlockSpec((_NC, bm, hdim // 2), lambda m: (0, m, 0)),
            pl.BlockSpec((_NC, bm, _LANES), lambda m: (0, m, 0)),
            pl.BlockSpec((1, hdim), lambda m: (0, 0)),
            pl.BlockSpec((hdim, hdim), lambda m: (0, 0)),
        ],
        out_specs=[
            pl.BlockSpec((bm, hdim), lambda m: (m, 0)),
            pl.BlockSpec((_NC, bm, hdim // 2), lambda m: (0, m, 0)),
        ],
        out_shape=[
            jax.ShapeDtypeStruct((n, hdim), F32),
            jax.ShapeDtypeStruct((_NC, n, hdim // 2), F32),
        ],
    )(h1, s3, cnt3, b1, W2)


def _k3_body(h_ref, s_ref, cnt_ref, b_ref, wc_ref, bc_ref, z_ref, y_ref):
    sfull = jnp.concatenate([s_ref[0], s_ref[1]], axis=1)
    invs, invd = _deg_terms(cnt_ref[...])
    z = invs * sfull + invd * h_ref[...] + b_ref[...]
    z_ref[...] = z
    y_ref[...] = jnp.dot(z, wc_ref[...], preferred_element_type=F32) + bc_ref[...]


def _k3(h2, s3, cnt3, b2, Wc, bc, n, bm):
    grid = (n // bm,)
    hdim = h2.shape[1]
    return pl.pallas_call(
        _k3_body,
        grid=grid,
        in_specs=[
            pl.BlockSpec((bm, hdim), lambda m: (m, 0)),
            pl.BlockSpec((_NC, bm, hdim // 2), lambda m: (0, m, 0)),
            pl.BlockSpec((_NC, bm, _LANES), lambda m: (0, m, 0)),
            pl.BlockSpec((1, hdim), lambda m: (0, 0)),
            pl.BlockSpec((hdim, 1), lambda m: (0, 0)),
            pl.BlockSpec((1, 1), lambda m: (0, 0)),
        ],
        out_specs=[
            pl.BlockSpec((bm, hdim), lambda m: (m, 0)),
            pl.BlockSpec((bm, 1), lambda m: (m, 0)),
        ],
        out_shape=[
            jax.ShapeDtypeStruct((n, hdim), F32),
            jax.ShapeDtypeStruct((n, 1), F32),
        ],
    )(h2, s3, cnt3, b2, Wc, bc)


def kernel(x, edge_index, W1, b1, W2, b2, Wc, bc):
    n, d = x.shape
    hdim = W1.shape[1]
    e = edge_index.shape[1]
    half = hdim // 2
    bm = 1000

    src = edge_index[0]
    dst = edge_index[1]
    ones = jnp.ones((40, _LANES), F32)
    zeros16 = jnp.zeros((n, _LANES), F32)
    zeros128 = jnp.zeros((n, half), F32)

    cnt = _deg_call(dst, ones, zeros16, n, e)
    cnt3 = cnt.reshape(_NC, n, _LANES)

    h1, u1 = _k1(x, W1, cnt3, n, bm)
    s1 = _edge_call(u1.reshape(_NC * n, half), src, dst, zeros128, n, e)
    h2, u2 = _k2(h1, s1.reshape(_NC, n, half), cnt3, b1.reshape(1, hdim), W2, n, bm)
    s2 = _edge_call(u2.reshape(_NC * n, half), src, dst, zeros128, n, e)
    z, y = _k3(h2, s2.reshape(_NC, n, half), cnt3, b2.reshape(1, hdim),
               Wc, bc.reshape(1, 1), n, bm)
    return (z, y)


# speedup vs baseline: 6.0564x; 6.0564x over previous
import functools

import jax
import jax.numpy as jnp
from jax import lax
from jax.experimental import pallas as pl
from jax.experimental.pallas import tpu as pltpu
from jax.experimental.pallas import tpu_sc as plsc

F32 = jnp.float32

_NC = 2
_NS = 16
_LANES = 16


def _deg_call(dst, ones, zeros16, n, e):
    w = ones.shape[1]
    chunk = 40
    per_tile = e // (_NC * _NS)
    nch = per_tile // chunk
    zrows = n // 10

    def body(dst_hbm, ones_hbm, zeros_hbm, out_hbm, idx_d, ones_v, acc_sh):
        c = lax.axis_index("c")
        s = lax.axis_index("s")

        @pl.when(s < 10)
        def _zero():
            pltpu.sync_copy(zeros_hbm.at[pl.ds(s * zrows, zrows)],
                            acc_sh.at[pl.ds(s * zrows, zrows)])

        pltpu.sync_copy(ones_hbm, ones_v)
        plsc.subcore_barrier()
        base = (c * _NS + s) * per_tile

        def step(i, carry):
            e0 = base + i * chunk
            pltpu.sync_copy(dst_hbm.at[pl.ds(e0, chunk)], idx_d)
            pltpu.sync_copy(ones_v, acc_sh.at[idx_d], add=True)
            return carry

        lax.fori_loop(0, nch, step, 0)
        plsc.subcore_barrier()

        @pl.when(s < 10)
        def _writeback():
            pltpu.sync_copy(acc_sh.at[pl.ds(s * zrows, zrows)],
                            out_hbm.at[pl.ds(c * n + s * zrows, zrows)])

    kfn = pl.kernel(
        body,
        out_type=jax.ShapeDtypeStruct((_NC * n, w), F32),
        mesh=plsc.VectorSubcoreMesh(core_axis_name="c", subcore_axis_name="s"),
        scratch_types=[
            pltpu.VMEM((chunk,), jnp.int32),
            pltpu.VMEM((chunk, w), F32),
            pltpu.VMEM_SHARED((n, w), F32),
        ],
    )
    return kfn(dst, ones, zeros16)


def _edge_call(u2, src, dst, zeros128, n, e):
    chunk = 80
    per_tile = e // _NS
    nch = per_tile // chunk
    zrows = n // 10
    half = u2.shape[1]

    def body(u_hbm, src_hbm, dst_hbm, zeros_hbm, out_hbm,
             idx_s, idx_d, rows_v, acc_sh, sem):
        c = lax.axis_index("c")
        s = lax.axis_index("s")

        @pl.when(s < 10)
        def _zero():
            pltpu.sync_copy(zeros_hbm.at[pl.ds(s * zrows, zrows)],
                            acc_sh.at[pl.ds(s * zrows, zrows)])

        plsc.subcore_barrier()
        base = s * per_tile
        row_off = c * n

        def step(i, carry):
            e0 = base + i * chunk
            pltpu.sync_copy(src_hbm.at[pl.ds(e0, chunk)], idx_s)
            pltpu.sync_copy(dst_hbm.at[pl.ds(e0, chunk)], idx_d)
            for j in range(chunk // _LANES):
                v = idx_s[pl.ds(j * _LANES, _LANES)]
                idx_s[pl.ds(j * _LANES, _LANES)] = v + row_off
            pltpu.async_copy(u_hbm.at[idx_s], rows_v, sem).wait()
            pltpu.sync_copy(rows_v, acc_sh.at[idx_d], add=True)
            return carry

        lax.fori_loop(0, nch, step, 0)
        plsc.subcore_barrier()

        @pl.when(s < 10)
        def _writeback():
            pltpu.sync_copy(acc_sh.at[pl.ds(s * zrows, zrows)],
                            out_hbm.at[pl.ds(row_off + s * zrows, zrows)])

    kfn = pl.kernel(
        body,
        out_type=jax.ShapeDtypeStruct((_NC * n, half), F32),
        mesh=plsc.VectorSubcoreMesh(core_axis_name="c", subcore_axis_name="s"),
        scratch_types=[
            pltpu.VMEM((chunk,), jnp.int32),
            pltpu.VMEM((chunk,), jnp.int32),
            pltpu.VMEM((chunk, half), F32),
            pltpu.VMEM_SHARED((n, half), F32),
            pltpu.SemaphoreType.DMA,
        ],
    )
    return kfn(u2, src, dst, zeros128)


def _deg_terms(cnt_blk):
    deg = 1.0 + cnt_blk[0, :, 0:1] + cnt_blk[1, :, 0:1]
    return lax.rsqrt(deg), 1.0 / deg


def _k1_body(x_ref, w_ref, cnt_ref, h_ref, u_ref):
    h = jnp.dot(x_ref[...], w_ref[...], preferred_element_type=F32)
    invs, _ = _deg_terms(cnt_ref[...])
    h_ref[...] = h
    u = invs * h
    hh = u.shape[1] // 2
    u_ref[0] = u[:, :hh]
    u_ref[1] = u[:, hh:]


def _k1(x, W1, cnt3, n, bm):
    grid = (n // bm,)
    d = x.shape[1]
    hdim = W1.shape[1]
    return pl.pallas_call(
        _k1_body,
        grid=grid,
        in_specs=[
            pl.BlockSpec((bm, d), lambda m: (m, 0)),
            pl.BlockSpec((d, hdim), lambda m: (0, 0)),
            pl.BlockSpec((_NC, bm, 128), lambda m: (0, m, 0)),
        ],
        out_specs=[
            pl.BlockSpec((bm, hdim), lambda m: (m, 0)),
            pl.BlockSpec((_NC, bm, hdim // 2), lambda m: (0, m, 0)),
        ],
        out_shape=[
            jax.ShapeDtypeStruct((n, hdim), F32),
            jax.ShapeDtypeStruct((_NC, n, hdim // 2), F32),
        ],
    )(x, W1, cnt3)


def _k2_body(h_ref, s_ref, cnt_ref, b_ref, w_ref, h2_ref, u_ref):
    sfull = jnp.concatenate([s_ref[0], s_ref[1]], axis=1)
    invs, invd = _deg_terms(cnt_ref[...])
    pre = invs * sfull + invd * h_ref[...] + b_ref[...]
    a = jnp.maximum(pre, 0.0)
    h2 = jnp.dot(a, w_ref[...], preferred_element_type=F32)
    h2_ref[...] = h2
    u = invs * h2
    hh = u.shape[1] // 2
    u_ref[0] = u[:, :hh]
    u_ref[1] = u[:, hh:]


def _k2(h1, s3, cnt3, b1, W2, n, bm):
    grid = (n // bm,)
    hdim = h1.shape[1]
    return pl.pallas_call(
        _k2_body,
        grid=grid,
        in_specs=[
            pl.BlockSpec((bm, hdim), lambda m: (m, 0)),
            pl.BlockSpec((_NC, bm, hdim // 2), lambda m: (0, m, 0)),
            pl.BlockSpec((_NC, bm, 128), lambda m: (0, m, 0)),
            pl.BlockSpec((1, hdim), lambda m: (0, 0)),
            pl.BlockSpec((hdim, hdim), lambda m: (0, 0)),
        ],
        out_specs=[
            pl.BlockSpec((bm, hdim), lambda m: (m, 0)),
            pl.BlockSpec((_NC, bm, hdim // 2), lambda m: (0, m, 0)),
        ],
        out_shape=[
            jax.ShapeDtypeStruct((n, hdim), F32),
            jax.ShapeDtypeStruct((_NC, n, hdim // 2), F32),
        ],
    )(h1, s3, cnt3, b1, W2)


def _k3_body(h_ref, s_ref, cnt_ref, b_ref, wc_ref, bc_ref, z_ref, y_ref):
    sfull = jnp.concatenate([s_ref[0], s_ref[1]], axis=1)
    invs, invd = _deg_terms(cnt_ref[...])
    z = invs * sfull + invd * h_ref[...] + b_ref[...]
    z_ref[...] = z
    y_ref[...] = jnp.dot(z, wc_ref[...], preferred_element_type=F32) + bc_ref[...]


def _k3(h2, s3, cnt3, b2, Wc, bc, n, bm):
    grid = (n // bm,)
    hdim = h2.shape[1]
    return pl.pallas_call(
        _k3_body,
        grid=grid,
        in_specs=[
            pl.BlockSpec((bm, hdim), lambda m: (m, 0)),
            pl.BlockSpec((_NC, bm, hdim // 2), lambda m: (0, m, 0)),
            pl.BlockSpec((_NC, bm, 128), lambda m: (0, m, 0)),
            pl.BlockSpec((1, hdim), lambda m: (0, 0)),
            pl.BlockSpec((hdim, 1), lambda m: (0, 0)),
            pl.BlockSpec((1, 1), lambda m: (0, 0)),
        ],
        out_specs=[
            pl.BlockSpec((bm, hdim), lambda m: (m, 0)),
            pl.BlockSpec((bm, 1), lambda m: (m, 0)),
        ],
        out_shape=[
            jax.ShapeDtypeStruct((n, hdim), F32),
            jax.ShapeDtypeStruct((n, 1), F32),
        ],
    )(h2, s3, cnt3, b2, Wc, bc)


def kernel(x, edge_index, W1, b1, W2, b2, Wc, bc):
    n, d = x.shape
    hdim = W1.shape[1]
    e = edge_index.shape[1]
    half = hdim // 2
    bm = 1000

    src = edge_index[0]
    dst = edge_index[1]
    ones = jnp.ones((40, half), F32)
    zeros128 = jnp.zeros((n, half), F32)

    cnt = _deg_call(dst, ones, zeros128, n, e)
    cnt3 = cnt.reshape(_NC, n, half)

    h1, u1 = _k1(x, W1, cnt3, n, bm)
    s1 = _edge_call(u1.reshape(_NC * n, half), src, dst, zeros128, n, e)
    h2, u2 = _k2(h1, s1.reshape(_NC, n, half), cnt3, b1.reshape(1, hdim), W2, n, bm)
    s2 = _edge_call(u2.reshape(_NC * n, half), src, dst, zeros128, n, e)
    z, y = _k3(h2, s2.reshape(_NC, n, half), cnt3, b2.reshape(1, hdim),
               Wc, bc.reshape(1, 1), n, bm)
    return (z, y)


# speedup vs baseline: 10.9410x; 1.8065x over previous
import functools

import jax
import jax.numpy as jnp
from jax import lax
from jax.experimental import pallas as pl
from jax.experimental.pallas import tpu as pltpu
from jax.experimental.pallas import tpu_sc as plsc

F32 = jnp.float32

_NC = 2
_NS = 16
_LANES = 16


def _deg_call(dst3, ones, zeros16, n, e):
    w = ones.shape[1]
    chunk = dst3.shape[2]
    nch = dst3.shape[1]
    zrows = n // 10

    def body(dst_hbm, ones_hbm, zeros_hbm, out_hbm, idx_d, ones_v, acc_sh, sem):
        c = lax.axis_index("c")
        s = lax.axis_index("s")

        @pl.when(s < 10)
        def _zero():
            pltpu.sync_copy(zeros_hbm.at[pl.ds(s * zrows, zrows)],
                            acc_sh.at[pl.ds(s * zrows, zrows)])

        pltpu.sync_copy(ones_hbm, ones_v)
        pltpu.sync_copy(dst_hbm.at[c * _NS + s], idx_d)
        plsc.subcore_barrier()

        def fire(i, carry):
            pltpu.async_copy(ones_v, acc_sh.at[idx_d.at[i]], sem, add=True)
            return carry

        lax.fori_loop(0, nch, fire, 0)

        def drain(i, carry):
            pltpu.make_async_copy(ones_v, acc_sh.at[idx_d.at[i]], sem).wait()
            return carry

        lax.fori_loop(0, nch, drain, 0)
        plsc.subcore_barrier()

        @pl.when(s < 10)
        def _writeback():
            pltpu.sync_copy(acc_sh.at[pl.ds(s * zrows, zrows)],
                            out_hbm.at[pl.ds(c * n + s * zrows, zrows)])

    kfn = pl.kernel(
        body,
        out_type=jax.ShapeDtypeStruct((_NC * n, w), F32),
        mesh=plsc.VectorSubcoreMesh(core_axis_name="c", subcore_axis_name="s"),
        scratch_types=[
            pltpu.VMEM((nch, chunk), jnp.int32),
            pltpu.VMEM((chunk, w), F32),
            pltpu.VMEM_SHARED((n, w), F32),
            pltpu.SemaphoreType.DMA,
        ],
    )
    return kfn(dst3, ones, zeros16)


def _edge_call(u2, src, dst3, zeros128, n, e):
    chunk = dst3.shape[2]
    nch = dst3.shape[1]
    per_tile = nch * chunk
    zrows = n // 10
    half = u2.shape[1]
    npair = nch // 2

    def body(u_hbm, src_hbm, dst_hbm, zeros_hbm, out_hbm,
             isrc, idst, rows0, rows1, acc_sh, gsem0, gsem1, ssem0, ssem1):
        c = lax.axis_index("c")
        s = lax.axis_index("s")

        @pl.when(s < 10)
        def _zero():
            pltpu.sync_copy(zeros_hbm.at[pl.ds(s * zrows, zrows)],
                            acc_sh.at[pl.ds(s * zrows, zrows)])

        pltpu.sync_copy(src_hbm.at[pl.ds(s * per_tile, per_tile)], isrc)
        pltpu.sync_copy(dst_hbm.at[s], idst)
        row_off = c * n

        def off(i, carry):
            isrc[pl.ds(i * _LANES, _LANES)] = (
                isrc[pl.ds(i * _LANES, _LANES)] + row_off)
            return carry

        lax.fori_loop(0, per_tile // _LANES, off, 0)
        plsc.subcore_barrier()

        rows = (rows0, rows1)
        gsem = (gsem0, gsem1)
        ssem = (ssem0, ssem1)

        def gather(i, b):
            pltpu.async_copy(u_hbm.at[isrc.at[pl.ds(i * chunk, chunk)]],
                             rows[b], gsem[b])

        def wait_gather(i, b):
            pltpu.make_async_copy(u_hbm.at[isrc.at[pl.ds(i * chunk, chunk)]],
                                  rows[b], gsem[b]).wait()

        def scatter(i, b):
            pltpu.async_copy(rows[b], acc_sh.at[idst.at[i]], ssem[b], add=True)

        def wait_scatter(i, b):
            pltpu.make_async_copy(rows[b], acc_sh.at[idst.at[i]], ssem[b]).wait()

        gather(0, 0)
        gather(1, 1)

        def pair(g, carry):
            i0 = 2 * g
            wait_gather(i0, 0)
            scatter(i0, 0)
            wait_gather(i0 + 1, 1)
            scatter(i0 + 1, 1)
            wait_scatter(i0, 0)
            gather(i0 + 2, 0)

            @pl.when(g < npair - 1)
            def _refill1():
                wait_scatter(i0 + 1, 1)
                gather(i0 + 3, 1)

            return carry

        lax.fori_loop(0, npair, pair, 0)
        wait_scatter(nch - 2, 1)
        wait_gather(nch - 1, 0)
        scatter(nch - 1, 0)
        wait_scatter(nch - 1, 0)
        plsc.subcore_barrier()

        @pl.when(s < 10)
        def _writeback():
            pltpu.sync_copy(acc_sh.at[pl.ds(s * zrows, zrows)],
                            out_hbm.at[pl.ds(row_off + s * zrows, zrows)])

    kfn = pl.kernel(
        body,
        out_type=jax.ShapeDtypeStruct((_NC * n, half), F32),
        mesh=plsc.VectorSubcoreMesh(core_axis_name="c", subcore_axis_name="s"),
        scratch_types=[
            pltpu.VMEM((nch * chunk,), jnp.int32),
            pltpu.VMEM((nch, chunk), jnp.int32),
            pltpu.VMEM((chunk, half), F32),
            pltpu.VMEM((chunk, half), F32),
            pltpu.VMEM_SHARED((n, half), F32),
            pltpu.SemaphoreType.DMA,
            pltpu.SemaphoreType.DMA,
            pltpu.SemaphoreType.DMA,
            pltpu.SemaphoreType.DMA,
        ],
    )
    return kfn(u2, src, dst3, zeros128)


def _deg_terms(cnt_blk):
    deg = 1.0 + cnt_blk[0, :, 0:1] + cnt_blk[1, :, 0:1]
    return lax.rsqrt(deg), 1.0 / deg


def _k1_body(x_ref, w_ref, cnt_ref, h_ref, u_ref):
    h = jnp.dot(x_ref[...], w_ref[...], preferred_element_type=F32)
    invs, _ = _deg_terms(cnt_ref[...])
    h_ref[...] = h
    u = invs * h
    hh = u.shape[1] // 2
    u_ref[0] = u[:, :hh]
    u_ref[1] = u[:, hh:]


def _k1(x, W1, cnt3, n, bm):
    grid = (n // bm,)
    d = x.shape[1]
    hdim = W1.shape[1]
    return pl.pallas_call(
        _k1_body,
        grid=grid,
        in_specs=[
            pl.BlockSpec((bm, d), lambda m: (m, 0)),
            pl.BlockSpec((d, hdim), lambda m: (0, 0)),
            pl.BlockSpec((_NC, bm, 128), lambda m: (0, m, 0)),
        ],
        out_specs=[
            pl.BlockSpec((bm, hdim), lambda m: (m, 0)),
            pl.BlockSpec((_NC, bm, hdim // 2), lambda m: (0, m, 0)),
        ],
        out_shape=[
            jax.ShapeDtypeStruct((n, hdim), F32),
            jax.ShapeDtypeStruct((_NC, n, hdim // 2), F32),
        ],
    )(x, W1, cnt3)


def _k2_body(h_ref, s_ref, cnt_ref, b_ref, w_ref, h2_ref, u_ref):
    sfull = jnp.concatenate([s_ref[0], s_ref[1]], axis=1)
    invs, invd = _deg_terms(cnt_ref[...])
    pre = invs * sfull + invd * h_ref[...] + b_ref[...]
    a = jnp.maximum(pre, 0.0)
    h2 = jnp.dot(a, w_ref[...], preferred_element_type=F32)
    h2_ref[...] = h2
    u = invs * h2
    hh = u.shape[1] // 2
    u_ref[0] = u[:, :hh]
    u_ref[1] = u[:, hh:]


def _k2(h1, s3, cnt3, b1, W2, n, bm):
    grid = (n // bm,)
    hdim = h1.shape[1]
    return pl.pallas_call(
        _k2_body,
        grid=grid,
        in_specs=[
            pl.BlockSpec((bm, hdim), lambda m: (m, 0)),
            pl.BlockSpec((_NC, bm, hdim // 2), lambda m: (0, m, 0)),
            pl.BlockSpec((_NC, bm, 128), lambda m: (0, m, 0)),
            pl.BlockSpec((1, hdim), lambda m: (0, 0)),
            pl.BlockSpec((hdim, hdim), lambda m: (0, 0)),
        ],
        out_specs=[
            pl.BlockSpec((bm, hdim), lambda m: (m, 0)),
            pl.BlockSpec((_NC, bm, hdim // 2), lambda m: (0, m, 0)),
        ],
        out_shape=[
            jax.ShapeDtypeStruct((n, hdim), F32),
            jax.ShapeDtypeStruct((_NC, n, hdim // 2), F32),
        ],
    )(h1, s3, cnt3, b1, W2)


def _k3_body(h_ref, s_ref, cnt_ref, b_ref, wc_ref, bc_ref, z_ref, y_ref):
    sfull = jnp.concatenate([s_ref[0], s_ref[1]], axis=1)
    invs, invd = _deg_terms(cnt_ref[...])
    z = invs * sfull + invd * h_ref[...] + b_ref[...]
    z_ref[...] = z
    y_ref[...] = jnp.dot(z, wc_ref[...], preferred_element_type=F32) + bc_ref[...]


def _k3(h2, s3, cnt3, b2, Wc, bc, n, bm):
    grid = (n // bm,)
    hdim = h2.shape[1]
    return pl.pallas_call(
        _k3_body,
        grid=grid,
        in_specs=[
            pl.BlockSpec((bm, hdim), lambda m: (m, 0)),
            pl.BlockSpec((_NC, bm, hdim // 2), lambda m: (0, m, 0)),
            pl.BlockSpec((_NC, bm, 128), lambda m: (0, m, 0)),
            pl.BlockSpec((1, hdim), lambda m: (0, 0)),
            pl.BlockSpec((hdim, 1), lambda m: (0, 0)),
            pl.BlockSpec((1, 1), lambda m: (0, 0)),
        ],
        out_specs=[
            pl.BlockSpec((bm, hdim), lambda m: (m, 0)),
            pl.BlockSpec((bm, 1), lambda m: (m, 0)),
        ],
        out_shape=[
            jax.ShapeDtypeStruct((n, hdim), F32),
            jax.ShapeDtypeStruct((n, 1), F32),
        ],
    )(h2, s3, cnt3, b2, Wc, bc)


def kernel(x, edge_index, W1, b1, W2, b2, Wc, bc):
    n, d = x.shape
    hdim = W1.shape[1]
    e = edge_index.shape[1]
    half = hdim // 2
    bm = 1000

    src = edge_index[0]
    dst = edge_index[1]
    dst3 = dst.reshape(_NS, e // (_NS * 80), 80)
    dst3_deg = dst.reshape(_NC * _NS, e // (_NC * _NS * 40), 40)
    ones = jnp.ones((40, half), F32)
    zeros128 = jnp.zeros((n, half), F32)

    cnt = _deg_call(dst3_deg, ones, zeros128, n, e)
    cnt3 = cnt.reshape(_NC, n, half)

    h1, u1 = _k1(x, W1, cnt3, n, bm)
    s1 = _edge_call(u1.reshape(_NC * n, half), src, dst3, zeros128, n, e)
    h2, u2 = _k2(h1, s1.reshape(_NC, n, half), cnt3, b1.reshape(1, hdim), W2, n, bm)
    s2 = _edge_call(u2.reshape(_NC * n, half), src, dst3, zeros128, n, e)
    z, y = _k3(h2, s2.reshape(_NC, n, half), cnt3, b2.reshape(1, hdim),
               Wc, bc.reshape(1, 1), n, bm)
    return (z, y)


# speedup vs baseline: 11.5646x; 1.0570x over previous
import functools

import jax
import jax.numpy as jnp
from jax import lax
from jax.experimental import pallas as pl
from jax.experimental.pallas import tpu as pltpu
from jax.experimental.pallas import tpu_sc as plsc

F32 = jnp.float32

_NC = 2
_NS = 16
_LANES = 16


def _deg_call(dst3, ones, zeros16, n, e):
    w = ones.shape[1]
    chunk = dst3.shape[2]
    nch = dst3.shape[1]
    zrows = n // 10

    def body(dst_hbm, ones_hbm, zeros_hbm, out_hbm, idx_d, ones_v, acc_sh, sem):
        c = lax.axis_index("c")
        s = lax.axis_index("s")

        @pl.when(s < 10)
        def _zero():
            pltpu.sync_copy(zeros_hbm.at[pl.ds(s * zrows, zrows)],
                            acc_sh.at[pl.ds(s * zrows, zrows)])

        pltpu.sync_copy(ones_hbm, ones_v)
        pltpu.sync_copy(dst_hbm.at[c * _NS + s], idx_d)
        plsc.subcore_barrier()

        def fire(i, carry):
            pltpu.async_copy(ones_v, acc_sh.at[idx_d.at[i]], sem, add=True)
            return carry

        lax.fori_loop(0, nch, fire, 0)

        def drain(i, carry):
            pltpu.make_async_copy(ones_v, acc_sh.at[idx_d.at[i]], sem).wait()
            return carry

        lax.fori_loop(0, nch, drain, 0)
        plsc.subcore_barrier()

        @pl.when(s < 10)
        def _writeback():
            pltpu.sync_copy(acc_sh.at[pl.ds(s * zrows, zrows)],
                            out_hbm.at[pl.ds(c * n + s * zrows, zrows)])

    kfn = pl.kernel(
        body,
        out_type=jax.ShapeDtypeStruct((_NC * n, w), F32),
        mesh=plsc.VectorSubcoreMesh(core_axis_name="c", subcore_axis_name="s"),
        scratch_types=[
            pltpu.VMEM((nch, chunk), jnp.int32),
            pltpu.VMEM((chunk, w), F32),
            pltpu.VMEM_SHARED((n, w), F32),
            pltpu.SemaphoreType.DMA,
        ],
    )
    return kfn(dst3, ones, zeros16)


def _edge_call(u2, srcs3, dst3, zeros128, n, e):
    chunk = dst3.shape[2]
    nch = dst3.shape[1]
    zrows = n // 10
    half = u2.shape[1]
    ngrp = nch // 3

    def body(u_hbm, src_hbm, dst_hbm, zeros_hbm, out_hbm,
             is0, is1, is2, idst, rows0, rows1, rows2, acc_sh,
             ise0, ise1, ise2, gse0, gse1, gse2, sse0, sse1, sse2):
        c = lax.axis_index("c")
        s = lax.axis_index("s")

        @pl.when(s < 10)
        def _zero():
            pltpu.sync_copy(zeros_hbm.at[pl.ds(s * zrows, zrows)],
                            acc_sh.at[pl.ds(s * zrows, zrows)])

        t = c * _NS + s
        pltpu.sync_copy(dst_hbm.at[s], idst)
        plsc.subcore_barrier()

        isb = (is0, is1, is2)
        rows = (rows0, rows1, rows2)
        isem = (ise0, ise1, ise2)
        gsem = (gse0, gse1, gse2)
        ssem = (sse0, sse1, sse2)

        def idxload(i, b):
            pltpu.async_copy(src_hbm.at[t].at[i], isb[b], isem[b])

        def wait_idx(i, b):
            pltpu.make_async_copy(src_hbm.at[t].at[i], isb[b], isem[b]).wait()

        def gather(i, b):
            pltpu.async_copy(u_hbm.at[isb[b]], rows[b], gsem[b])

        def wait_gather(i, b):
            pltpu.make_async_copy(u_hbm.at[isb[b]], rows[b], gsem[b]).wait()

        def scatter(i, b):
            pltpu.async_copy(rows[b], acc_sh.at[idst.at[i]], ssem[b], add=True)

        def wait_scatter(i, b):
            pltpu.make_async_copy(rows[b], acc_sh.at[idst.at[i]], ssem[b]).wait()

        idxload(0, 0)
        idxload(1, 1)
        wait_idx(0, 0)
        gather(0, 0)
        idxload(2, 2)
        wait_idx(1, 1)
        gather(1, 1)
        wait_gather(0, 0)
        scatter(0, 0)
        idxload(3, 0)
        wait_idx(2, 2)
        gather(2, 2)
        wait_gather(1, 1)
        scatter(1, 1)
        idxload(4, 1)

        def grp(g, carry):
            for b in range(3):
                i = 3 * g + b
                bp = (b + 2) % 3
                wait_idx(i, b)
                wait_scatter(i - 3, b)
                gather(i, b)
                wait_gather(i - 1, bp)
                scatter(i - 1, bp)
                if b == 0:
                    idxload(i + 2, bp)
                else:
                    @pl.when(g < ngrp - 1)
                    def _pf():
                        idxload(i + 2, bp)
            return carry

        lax.fori_loop(1, ngrp, grp, 0)
        wait_gather(nch - 1, (nch - 1) % 3)
        scatter(nch - 1, (nch - 1) % 3)
        wait_scatter(nch - 3, (nch - 3) % 3)
        wait_scatter(nch - 2, (nch - 2) % 3)
        wait_scatter(nch - 1, (nch - 1) % 3)
        plsc.subcore_barrier()

        @pl.when(s < 10)
        def _writeback():
            pltpu.sync_copy(acc_sh.at[pl.ds(s * zrows, zrows)],
                            out_hbm.at[pl.ds(c * n + s * zrows, zrows)])

    kfn = pl.kernel(
        body,
        out_type=jax.ShapeDtypeStruct((_NC * n, half), F32),
        mesh=plsc.VectorSubcoreMesh(core_axis_name="c", subcore_axis_name="s"),
        scratch_types=[
            pltpu.VMEM((chunk,), jnp.int32),
            pltpu.VMEM((chunk,), jnp.int32),
            pltpu.VMEM((chunk,), jnp.int32),
            pltpu.VMEM((nch, chunk), jnp.int32),
            pltpu.VMEM((chunk, half), F32),
            pltpu.VMEM((chunk, half), F32),
            pltpu.VMEM((chunk, half), F32),
            pltpu.VMEM_SHARED((n + 8, half), F32),
            pltpu.SemaphoreType.DMA,
            pltpu.SemaphoreType.DMA,
            pltpu.SemaphoreType.DMA,
            pltpu.SemaphoreType.DMA,
            pltpu.SemaphoreType.DMA,
            pltpu.SemaphoreType.DMA,
            pltpu.SemaphoreType.DMA,
            pltpu.SemaphoreType.DMA,
            pltpu.SemaphoreType.DMA,
        ],
    )
    return kfn(u2, srcs3, dst3, zeros128)


def _deg_terms(cnt_blk):
    deg = 1.0 + cnt_blk[0, :, 0:1] + cnt_blk[1, :, 0:1]
    return lax.rsqrt(deg), 1.0 / deg


def _k1_body(x_ref, w_ref, cnt_ref, h_ref, u_ref):
    h = jnp.dot(x_ref[...], w_ref[...], preferred_element_type=F32)
    invs, _ = _deg_terms(cnt_ref[...])
    h_ref[...] = h
    u = invs * h
    hh = u.shape[1] // 2
    u_ref[0] = u[:, :hh]
    u_ref[1] = u[:, hh:]


def _k1(x, W1, cnt3, n, bm):
    grid = (n // bm,)
    d = x.shape[1]
    hdim = W1.shape[1]
    return pl.pallas_call(
        _k1_body,
        grid=grid,
        in_specs=[
            pl.BlockSpec((bm, d), lambda m: (m, 0)),
            pl.BlockSpec((d, hdim), lambda m: (0, 0)),
            pl.BlockSpec((_NC, bm, 128), lambda m: (0, m, 0)),
        ],
        out_specs=[
            pl.BlockSpec((bm, hdim), lambda m: (m, 0)),
            pl.BlockSpec((_NC, bm, hdim // 2), lambda m: (0, m, 0)),
        ],
        out_shape=[
            jax.ShapeDtypeStruct((n, hdim), F32),
            jax.ShapeDtypeStruct((_NC, n, hdim // 2), F32),
        ],
    )(x, W1, cnt3)


def _k2_body(h_ref, s_ref, cnt_ref, b_ref, w_ref, h2_ref, u_ref):
    sfull = jnp.concatenate([s_ref[0], s_ref[1]], axis=1)
    invs, invd = _deg_terms(cnt_ref[...])
    pre = invs * sfull + invd * h_ref[...] + b_ref[...]
    a = jnp.maximum(pre, 0.0)
    h2 = jnp.dot(a, w_ref[...], preferred_element_type=F32)
    h2_ref[...] = h2
    u = invs * h2
    hh = u.shape[1] // 2
    u_ref[0] = u[:, :hh]
    u_ref[1] = u[:, hh:]


def _k2(h1, s3, cnt3, b1, W2, n, bm):
    grid = (n // bm,)
    hdim = h1.shape[1]
    return pl.pallas_call(
        _k2_body,
        grid=grid,
        in_specs=[
            pl.BlockSpec((bm, hdim), lambda m: (m, 0)),
            pl.BlockSpec((_NC, bm, hdim // 2), lambda m: (0, m, 0)),
            pl.BlockSpec((_NC, bm, 128), lambda m: (0, m, 0)),
            pl.BlockSpec((1, hdim), lambda m: (0, 0)),
            pl.BlockSpec((hdim, hdim), lambda m: (0, 0)),
        ],
        out_specs=[
            pl.BlockSpec((bm, hdim), lambda m: (m, 0)),
            pl.BlockSpec((_NC, bm, hdim // 2), lambda m: (0, m, 0)),
        ],
        out_shape=[
            jax.ShapeDtypeStruct((n, hdim), F32),
            jax.ShapeDtypeStruct((_NC, n, hdim // 2), F32),
        ],
    )(h1, s3, cnt3, b1, W2)


def _k3_body(h_ref, s_ref, cnt_ref, b_ref, wc_ref, bc_ref, z_ref, y_ref):
    sfull = jnp.concatenate([s_ref[0], s_ref[1]], axis=1)
    invs, invd = _deg_terms(cnt_ref[...])
    z = invs * sfull + invd * h_ref[...] + b_ref[...]
    z_ref[...] = z
    y_ref[...] = jnp.dot(z, wc_ref[...], preferred_element_type=F32) + bc_ref[...]


def _k3(h2, s3, cnt3, b2, Wc, bc, n, bm):
    grid = (n // bm,)
    hdim = h2.shape[1]
    return pl.pallas_call(
        _k3_body,
        grid=grid,
        in_specs=[
            pl.BlockSpec((bm, hdim), lambda m: (m, 0)),
            pl.BlockSpec((_NC, bm, hdim // 2), lambda m: (0, m, 0)),
            pl.BlockSpec((_NC, bm, 128), lambda m: (0, m, 0)),
            pl.BlockSpec((1, hdim), lambda m: (0, 0)),
            pl.BlockSpec((hdim, 1), lambda m: (0, 0)),
            pl.BlockSpec((1, 1), lambda m: (0, 0)),
        ],
        out_specs=[
            pl.BlockSpec((bm, hdim), lambda m: (m, 0)),
            pl.BlockSpec((bm, 1), lambda m: (m, 0)),
        ],
        out_shape=[
            jax.ShapeDtypeStruct((n, hdim), F32),
            jax.ShapeDtypeStruct((n, 1), F32),
        ],
    )(h2, s3, cnt3, b2, Wc, bc)


def kernel(x, edge_index, W1, b1, W2, b2, Wc, bc):
    n, d = x.shape
    hdim = W1.shape[1]
    e = edge_index.shape[1]
    half = hdim // 2
    bm = 1000

    src = edge_index[0]
    dst = edge_index[1]
    dst3_deg = dst.reshape(_NC * _NS, e // (_NC * _NS * 40), 40)
    ones = jnp.ones((40, half), F32)
    zeros128 = jnp.zeros((n, half), F32)

    ept = e // _NS
    pad = (-ept) % 240
    src_r = jnp.pad(src.reshape(_NS, ept), ((0, 0), (0, pad)))
    srcs3 = jnp.stack([src_r, src_r + n]).reshape(_NC * _NS, (ept + pad) // 80, 80)
    dst3 = jnp.pad(dst.reshape(_NS, ept), ((0, 0), (0, pad)),
                   constant_values=n).reshape(_NS, (ept + pad) // 80, 80)

    cnt = _deg_call(dst3_deg, ones, zeros128, n, e)
    cnt3 = cnt.reshape(_NC, n, half)

    h1, u1 = _k1(x, W1, cnt3, n, bm)
    s1 = _edge_call(u1.reshape(_NC * n, half), srcs3, dst3, zeros128, n, e)
    h2, u2 = _k2(h1, s1.reshape(_NC, n, half), cnt3, b1.reshape(1, hdim), W2, n, bm)
    s2 = _edge_call(u2.reshape(_NC * n, half), srcs3, dst3, zeros128, n, e)
    z, y = _k3(h2, s2.reshape(_NC, n, half), cnt3, b2.reshape(1, hdim),
               Wc, bc.reshape(1, 1), n, bm)
    return (z, y)


# speedup vs baseline: 11.6024x; 1.0033x over previous
import functools

import jax
import jax.numpy as jnp
from jax import lax
from jax.experimental import pallas as pl
from jax.experimental.pallas import tpu as pltpu
from jax.experimental.pallas import tpu_sc as plsc

F32 = jnp.float32

_NC = 2
_NS = 16
_LANES = 16


def _deg_call(dst3, ones, zeros16, n, e):
    w = ones.shape[1]
    chunk = dst3.shape[2]
    nch = dst3.shape[1]
    zrows = n // 10

    def body(dst_hbm, ones_hbm, zeros_hbm, out_hbm, idx_d, ones_v, acc_sh, sem):
        c = lax.axis_index("c")
        s = lax.axis_index("s")

        @pl.when(s < 10)
        def _zero():
            pltpu.sync_copy(zeros_hbm.at[pl.ds(s * zrows, zrows)],
                            acc_sh.at[pl.ds(s * zrows, zrows)])

        pltpu.sync_copy(ones_hbm, ones_v)
        pltpu.sync_copy(dst_hbm.at[c * _NS + s], idx_d)
        plsc.subcore_barrier()

        def fire(i, carry):
            pltpu.async_copy(ones_v, acc_sh.at[idx_d.at[i]], sem, add=True)
            return carry

        lax.fori_loop(0, nch, fire, 0)

        def drain(i, carry):
            pltpu.make_async_copy(ones_v, acc_sh.at[idx_d.at[i]], sem).wait()
            return carry

        lax.fori_loop(0, nch, drain, 0)
        plsc.subcore_barrier()

        @pl.when(s < 10)
        def _writeback():
            pltpu.sync_copy(acc_sh.at[pl.ds(s * zrows, zrows)],
                            out_hbm.at[pl.ds(c * n + s * zrows, zrows)])

    kfn = pl.kernel(
        body,
        out_type=jax.ShapeDtypeStruct((_NC * n, w), F32),
        mesh=plsc.VectorSubcoreMesh(core_axis_name="c", subcore_axis_name="s"),
        scratch_types=[
            pltpu.VMEM((nch, chunk), jnp.int32),
            pltpu.VMEM((chunk, w), F32),
            pltpu.VMEM_SHARED((n, w), F32),
            pltpu.SemaphoreType.DMA,
        ],
    )
    return kfn(dst3, ones, zeros16)


def _edge_call(u2, srcs3, dst3, zeros128, n, e):
    chunk = dst3.shape[2]
    nch = dst3.shape[1]
    zrows = n // 10
    half = u2.shape[1]
    ngrp = nch // 3

    def body(u_hbm, src_hbm, dst_hbm, zeros_hbm, out_hbm,
             is0, is1, is2, idst, rows0, rows1, rows2, acc_sh,
             ise0, ise1, ise2, gse0, gse1, gse2, sse0, sse1, sse2):
        c = lax.axis_index("c")
        s = lax.axis_index("s")

        @pl.when(s < 10)
        def _zero():
            pltpu.sync_copy(zeros_hbm.at[pl.ds(s * zrows, zrows)],
                            acc_sh.at[pl.ds(s * zrows, zrows)])

        t = c * _NS + s
        pltpu.sync_copy(dst_hbm.at[s], idst)
        plsc.subcore_barrier()

        isb = (is0, is1, is2)
        rows = (rows0, rows1, rows2)
        isem = (ise0, ise1, ise2)
        gsem = (gse0, gse1, gse2)
        ssem = (sse0, sse1, sse2)

        def idxload(i, b):
            pltpu.async_copy(src_hbm.at[t].at[i], isb[b], isem[b])

        def wait_idx(i, b):
            pltpu.make_async_copy(src_hbm.at[t].at[i], isb[b], isem[b]).wait()

        def gather(i, b):
            pltpu.async_copy(u_hbm.at[isb[b]], rows[b], gsem[b])

        def wait_gather(i, b):
            pltpu.make_async_copy(u_hbm.at[isb[b]], rows[b], gsem[b]).wait()

        def scatter(i, b):
            pltpu.async_copy(rows[b], acc_sh.at[idst.at[i]], ssem[b], add=True)

        def wait_scatter(i, b):
            pltpu.make_async_copy(rows[b], acc_sh.at[idst.at[i]], ssem[b]).wait()

        idxload(0, 0)
        idxload(1, 1)
        wait_idx(0, 0)
        gather(0, 0)
        idxload(2, 2)
        wait_idx(1, 1)
        gather(1, 1)
        wait_gather(0, 0)
        scatter(0, 0)
        idxload(3, 0)
        wait_idx(2, 2)
        gather(2, 2)
        wait_gather(1, 1)
        scatter(1, 1)
        idxload(4, 1)

        def grp(g, carry):
            for b in range(3):
                i = 3 * g + b
                bp = (b + 2) % 3
                wait_idx(i, b)
                wait_scatter(i - 3, b)
                gather(i, b)
                wait_gather(i - 1, bp)
                scatter(i - 1, bp)
                if b == 0:
                    idxload(i + 2, bp)
                else:
                    @pl.when(g < ngrp - 1)
                    def _pf():
                        idxload(i + 2, bp)
            return carry

        lax.fori_loop(1, ngrp, grp, 0)
        wait_gather(nch - 1, (nch - 1) % 3)
        scatter(nch - 1, (nch - 1) % 3)
        wait_scatter(nch - 3, (nch - 3) % 3)
        wait_scatter(nch - 2, (nch - 2) % 3)
        wait_scatter(nch - 1, (nch - 1) % 3)
        plsc.subcore_barrier()

        @pl.when(s < 10)
        def _writeback():
            pltpu.sync_copy(acc_sh.at[pl.ds(s * zrows, zrows)],
                            out_hbm.at[pl.ds(c * n + s * zrows, zrows)])

    kfn = pl.kernel(
        body,
        out_type=jax.ShapeDtypeStruct((_NC * n, half), F32),
        mesh=plsc.VectorSubcoreMesh(core_axis_name="c", subcore_axis_name="s"),
        scratch_types=[
            pltpu.VMEM((chunk,), jnp.int32),
            pltpu.VMEM((chunk,), jnp.int32),
            pltpu.VMEM((chunk,), jnp.int32),
            pltpu.VMEM((nch, chunk), jnp.int32),
            pltpu.VMEM((chunk, half), F32),
            pltpu.VMEM((chunk, half), F32),
            pltpu.VMEM((chunk, half), F32),
            pltpu.VMEM_SHARED((n + 8, half), F32),
            pltpu.SemaphoreType.DMA,
            pltpu.SemaphoreType.DMA,
            pltpu.SemaphoreType.DMA,
            pltpu.SemaphoreType.DMA,
            pltpu.SemaphoreType.DMA,
            pltpu.SemaphoreType.DMA,
            pltpu.SemaphoreType.DMA,
            pltpu.SemaphoreType.DMA,
            pltpu.SemaphoreType.DMA,
        ],
    )
    return kfn(u2, srcs3, dst3, zeros128)


def _deg_terms(cnt_blk):
    deg = 1.0 + cnt_blk[0, :, 0:1] + cnt_blk[1, :, 0:1]
    return lax.rsqrt(deg), 1.0 / deg


def _k1a_body(x_ref, w_ref, h_ref):
    h_ref[...] = jnp.dot(x_ref[...], w_ref[...], preferred_element_type=F32)


def _k1a(x, W1, n, bm):
    grid = (n // bm,)
    d = x.shape[1]
    hdim = W1.shape[1]
    return pl.pallas_call(
        _k1a_body,
        grid=grid,
        in_specs=[
            pl.BlockSpec((bm, d), lambda m: (m, 0)),
            pl.BlockSpec((d, hdim), lambda m: (0, 0)),
        ],
        out_specs=pl.BlockSpec((bm, hdim), lambda m: (m, 0)),
        out_shape=jax.ShapeDtypeStruct((n, hdim), F32),
    )(x, W1)


def _k1b_body(h_ref, cnt_ref, u_ref):
    invs, _ = _deg_terms(cnt_ref[...])
    u = invs * h_ref[...]
    hh = u.shape[1] // 2
    u_ref[0] = u[:, :hh]
    u_ref[1] = u[:, hh:]


def _k1b(h1, cnt3, n, bm):
    grid = (n // bm,)
    hdim = h1.shape[1]
    return pl.pallas_call(
        _k1b_body,
        grid=grid,
        in_specs=[
            pl.BlockSpec((bm, hdim), lambda m: (m, 0)),
            pl.BlockSpec((_NC, bm, 128), lambda m: (0, m, 0)),
        ],
        out_specs=pl.BlockSpec((_NC, bm, hdim // 2), lambda m: (0, m, 0)),
        out_shape=jax.ShapeDtypeStruct((_NC, n, hdim // 2), F32),
    )(h1, cnt3)


def _k2_body(h_ref, s_ref, cnt_ref, b_ref, w_ref, h2_ref, u_ref):
    sfull = jnp.concatenate([s_ref[0], s_ref[1]], axis=1)
    invs, invd = _deg_terms(cnt_ref[...])
    pre = invs * sfull + invd * h_ref[...] + b_ref[...]
    a = jnp.maximum(pre, 0.0)
    h2 = jnp.dot(a, w_ref[...], preferred_element_type=F32)
    h2_ref[...] = h2
    u = invs * h2
    hh = u.shape[1] // 2
    u_ref[0] = u[:, :hh]
    u_ref[1] = u[:, hh:]


def _k2(h1, s3, cnt3, b1, W2, n, bm):
    grid = (n // bm,)
    hdim = h1.shape[1]
    return pl.pallas_call(
        _k2_body,
        grid=grid,
        in_specs=[
            pl.BlockSpec((bm, hdim), lambda m: (m, 0)),
            pl.BlockSpec((_NC, bm, hdim // 2), lambda m: (0, m, 0)),
            pl.BlockSpec((_NC, bm, 128), lambda m: (0, m, 0)),
            pl.BlockSpec((1, hdim), lambda m: (0, 0)),
            pl.BlockSpec((hdim, hdim), lambda m: (0, 0)),
        ],
        out_specs=[
            pl.BlockSpec((bm, hdim), lambda m: (m, 0)),
            pl.BlockSpec((_NC, bm, hdim // 2), lambda m: (0, m, 0)),
        ],
        out_shape=[
            jax.ShapeDtypeStruct((n, hdim), F32),
            jax.ShapeDtypeStruct((_NC, n, hdim // 2), F32),
        ],
    )(h1, s3, cnt3, b1, W2)


def _k3_body(h_ref, s_ref, cnt_ref, b_ref, wc_ref, bc_ref, z_ref, y_ref):
    sfull = jnp.concatenate([s_ref[0], s_ref[1]], axis=1)
    invs, invd = _deg_terms(cnt_ref[...])
    z = invs * sfull + invd * h_ref[...] + b_ref[...]
    z_ref[...] = z
    y_ref[...] = jnp.dot(z, wc_ref[...], preferred_element_type=F32) + bc_ref[...]


def _k3(h2, s3, cnt3, b2, Wc, bc, n, bm):
    grid = (n // bm,)
    hdim = h2.shape[1]
    return pl.pallas_call(
        _k3_body,
        grid=grid,
        in_specs=[
            pl.BlockSpec((bm, hdim), lambda m: (m, 0)),
            pl.BlockSpec((_NC, bm, hdim // 2), lambda m: (0, m, 0)),
            pl.BlockSpec((_NC, bm, 128), lambda m: (0, m, 0)),
            pl.BlockSpec((1, hdim), lambda m: (0, 0)),
            pl.BlockSpec((hdim, 1), lambda m: (0, 0)),
            pl.BlockSpec((1, 1), lambda m: (0, 0)),
        ],
        out_specs=[
            pl.BlockSpec((bm, hdim), lambda m: (m, 0)),
            pl.BlockSpec((bm, 1), lambda m: (m, 0)),
        ],
        out_shape=[
            jax.ShapeDtypeStruct((n, hdim), F32),
            jax.ShapeDtypeStruct((n, 1), F32),
        ],
    )(h2, s3, cnt3, b2, Wc, bc)


def kernel(x, edge_index, W1, b1, W2, b2, Wc, bc):
    n, d = x.shape
    hdim = W1.shape[1]
    e = edge_index.shape[1]
    half = hdim // 2
    bm = 1000

    src = edge_index[0]
    dst = edge_index[1]
    dst3_deg = dst.reshape(_NC * _NS, e // (_NC * _NS * 40), 40)
    ones = jnp.ones((40, half), F32)
    zeros128 = jnp.zeros((n, half), F32)

    ept = e // _NS
    pad = (-ept) % 240
    src_r = jnp.pad(src.reshape(_NS, ept), ((0, 0), (0, pad)))
    srcs3 = jnp.stack([src_r, src_r + n]).reshape(_NC * _NS, (ept + pad) // 80, 80)
    dst3 = jnp.pad(dst.reshape(_NS, ept), ((0, 0), (0, pad)),
                   constant_values=n).reshape(_NS, (ept + pad) // 80, 80)

    cnt = _deg_call(dst3_deg, ones, zeros128, n, e)
    cnt3 = cnt.reshape(_NC, n, half)

    h1 = _k1a(x, W1, n, bm)
    u1 = _k1b(h1, cnt3, n, bm)
    s1 = _edge_call(u1.reshape(_NC * n, half), srcs3, dst3, zeros128, n, e)
    h2, u2 = _k2(h1, s1.reshape(_NC, n, half), cnt3, b1.reshape(1, hdim), W2, n, bm)
    s2 = _edge_call(u2.reshape(_NC * n, half), srcs3, dst3, zeros128, n, e)
    z, y = _k3(h2, s2.reshape(_NC, n, half), cnt3, b2.reshape(1, hdim),
               Wc, bc.reshape(1, 1), n, bm)
    return (z, y)


# speedup vs baseline: 11.6881x; 1.0074x over previous
import functools

import jax
import jax.numpy as jnp
from jax import lax
from jax.experimental import pallas as pl
from jax.experimental.pallas import tpu as pltpu
from jax.experimental.pallas import tpu_sc as plsc

F32 = jnp.float32

_NC = 2
_NS = 16
_LANES = 16


def _deg_call(dst3, ones, zeros16, n, e):
    w = ones.shape[1]
    chunk = dst3.shape[2]
    nch = dst3.shape[1]
    zrows = n // 10

    def body(dst_hbm, ones_hbm, zeros_hbm, out_hbm, idx_d, ones_v, acc_sh, sem):
        c = lax.axis_index("c")
        s = lax.axis_index("s")

        @pl.when(s < 10)
        def _zero():
            pltpu.sync_copy(zeros_hbm.at[pl.ds(s * zrows, zrows)],
                            acc_sh.at[pl.ds(s * zrows, zrows)])

        pltpu.sync_copy(ones_hbm, ones_v)
        pltpu.sync_copy(dst_hbm.at[c * _NS + s], idx_d)
        plsc.subcore_barrier()

        def fire(i, carry):
            pltpu.async_copy(ones_v, acc_sh.at[idx_d.at[i]], sem, add=True)
            return carry

        lax.fori_loop(0, nch, fire, 0)

        def drain(i, carry):
            pltpu.make_async_copy(ones_v, acc_sh.at[idx_d.at[i]], sem).wait()
            return carry

        lax.fori_loop(0, nch, drain, 0)
        plsc.subcore_barrier()

        @pl.when(s < 10)
        def _writeback():
            pltpu.sync_copy(acc_sh.at[pl.ds(s * zrows, zrows)],
                            out_hbm.at[pl.ds(c * n + s * zrows, zrows)])

    kfn = pl.kernel(
        body,
        out_type=jax.ShapeDtypeStruct((_NC * n, w), F32),
        mesh=plsc.VectorSubcoreMesh(core_axis_name="c", subcore_axis_name="s"),
        scratch_types=[
            pltpu.VMEM((nch, chunk), jnp.int32),
            pltpu.VMEM((chunk, w), F32),
            pltpu.VMEM_SHARED((n, w), F32),
            pltpu.SemaphoreType.DMA,
        ],
    )
    return kfn(dst3, ones, zeros16)


def _edge_call(u2, srcs3, dst3, zeros128, n, e):
    chunk = dst3.shape[2]
    nch = dst3.shape[1]
    zrows = n // 10
    half = u2.shape[1]
    ngrp = nch // 3

    def body(u_hbm, src_hbm, dst_hbm, zeros_hbm, out_hbm,
             is0, is1, is2, idst, rows0, rows1, rows2, acc_sh,
             ise0, ise1, ise2, gse0, gse1, gse2, sse0, sse1, sse2):
        c = lax.axis_index("c")
        s = lax.axis_index("s")

        @pl.when(s < 10)
        def _zero():
            pltpu.sync_copy(zeros_hbm.at[pl.ds(s * zrows, zrows)],
                            acc_sh.at[pl.ds(s * zrows, zrows)])

        t = c * _NS + s
        pltpu.sync_copy(dst_hbm.at[s], idst)
        plsc.subcore_barrier()

        isb = (is0, is1, is2)
        rows = (rows0, rows1, rows2)
        isem = (ise0, ise1, ise2)
        gsem = (gse0, gse1, gse2)
        ssem = (sse0, sse1, sse2)

        def idxload(i, b):
            pltpu.async_copy(src_hbm.at[t].at[i], isb[b], isem[b])

        def wait_idx(i, b):
            pltpu.make_async_copy(src_hbm.at[t].at[i], isb[b], isem[b]).wait()

        def gather(i, b):
            pltpu.async_copy(u_hbm.at[isb[b]], rows[b], gsem[b])

        def wait_gather(i, b):
            pltpu.make_async_copy(u_hbm.at[isb[b]], rows[b], gsem[b]).wait()

        def scatter(i, b):
            pltpu.async_copy(rows[b], acc_sh.at[idst.at[i]], ssem[b], add=True)

        def wait_scatter(i, b):
            pltpu.make_async_copy(rows[b], acc_sh.at[idst.at[i]], ssem[b]).wait()

        idxload(0, 0)
        idxload(1, 1)
        wait_idx(0, 0)
        gather(0, 0)
        idxload(2, 2)
        wait_idx(1, 1)
        gather(1, 1)
        wait_gather(0, 0)
        scatter(0, 0)
        idxload(3, 0)
        wait_idx(2, 2)
        gather(2, 2)
        wait_gather(1, 1)
        scatter(1, 1)
        idxload(4, 1)

        def grp(g, carry):
            for b in range(3):
                i = 3 * g + b
                bp = (b + 2) % 3
                wait_idx(i, b)
                wait_scatter(i - 3, b)
                gather(i, b)
                wait_gather(i - 1, bp)
                scatter(i - 1, bp)
                if b == 0:
                    idxload(i + 2, bp)
                else:
                    @pl.when(g < ngrp - 1)
                    def _pf():
                        idxload(i + 2, bp)
            return carry

        lax.fori_loop(1, ngrp, grp, 0)
        wait_gather(nch - 1, (nch - 1) % 3)
        scatter(nch - 1, (nch - 1) % 3)
        wait_scatter(nch - 3, (nch - 3) % 3)
        wait_scatter(nch - 2, (nch - 2) % 3)
        wait_scatter(nch - 1, (nch - 1) % 3)
        plsc.subcore_barrier()

        @pl.when(s < 10)
        def _writeback():
            pltpu.sync_copy(acc_sh.at[pl.ds(s * zrows, zrows)],
                            out_hbm.at[pl.ds(c * n + s * zrows, zrows)])

    kfn = pl.kernel(
        body,
        out_type=jax.ShapeDtypeStruct((_NC * n, half), F32),
        mesh=plsc.VectorSubcoreMesh(core_axis_name="c", subcore_axis_name="s"),
        scratch_types=[
            pltpu.VMEM((chunk,), jnp.int32),
            pltpu.VMEM((chunk,), jnp.int32),
            pltpu.VMEM((chunk,), jnp.int32),
            pltpu.VMEM((nch, chunk), jnp.int32),
            pltpu.VMEM((chunk, half), F32),
            pltpu.VMEM((chunk, half), F32),
            pltpu.VMEM((chunk, half), F32),
            pltpu.VMEM_SHARED((n + 8, half), F32),
            pltpu.SemaphoreType.DMA,
            pltpu.SemaphoreType.DMA,
            pltpu.SemaphoreType.DMA,
            pltpu.SemaphoreType.DMA,
            pltpu.SemaphoreType.DMA,
            pltpu.SemaphoreType.DMA,
            pltpu.SemaphoreType.DMA,
            pltpu.SemaphoreType.DMA,
            pltpu.SemaphoreType.DMA,
        ],
    )
    return kfn(u2, srcs3, dst3, zeros128)


def _deg_terms(cnt_blk):
    deg = 1.0 + cnt_blk[0, :, 0:1] + cnt_blk[1, :, 0:1]
    return lax.rsqrt(deg), 1.0 / deg


def _k1a_body(x_ref, w_ref, h_ref):
    h_ref[...] = jnp.dot(x_ref[...], w_ref[...], preferred_element_type=F32)


def _k1a(x, W1, n, bm):
    grid = (n // bm,)
    d = x.shape[1]
    hdim = W1.shape[1]
    return pl.pallas_call(
        _k1a_body,
        grid=grid,
        in_specs=[
            pl.BlockSpec((bm, d), lambda m: (m, 0)),
            pl.BlockSpec((d, hdim), lambda m: (0, 0)),
        ],
        out_specs=pl.BlockSpec((bm, hdim), lambda m: (m, 0)),
        out_shape=jax.ShapeDtypeStruct((n, hdim), F32),
    )(x, W1)


def _k1b_body(h_ref, cnt_ref, u_ref):
    invs, _ = _deg_terms(cnt_ref[...])
    u = invs * h_ref[...]
    hh = u.shape[1] // 2
    u_ref[0] = u[:, :hh]
    u_ref[1] = u[:, hh:]


def _k1b(h1, cnt3, n, bm):
    grid = (n // bm,)
    hdim = h1.shape[1]
    return pl.pallas_call(
        _k1b_body,
        grid=grid,
        in_specs=[
            pl.BlockSpec((bm, hdim), lambda m: (m, 0)),
            pl.BlockSpec((_NC, bm, 128), lambda m: (0, m, 0)),
        ],
        out_specs=pl.BlockSpec((_NC, bm, hdim // 2), lambda m: (0, m, 0)),
        out_shape=jax.ShapeDtypeStruct((_NC, n, hdim // 2), F32),
    )(h1, cnt3)


def _k2_body(h_ref, s_ref, cnt_ref, b_ref, w_ref, h2_ref, u_ref):
    sfull = jnp.concatenate([s_ref[0], s_ref[1]], axis=1)
    invs, invd = _deg_terms(cnt_ref[...])
    pre = invs * sfull + invd * h_ref[...] + b_ref[...]
    a = jnp.maximum(pre, 0.0)
    h2 = jnp.dot(a, w_ref[...], preferred_element_type=F32)
    h2_ref[...] = h2
    u = invs * h2
    hh = u.shape[1] // 2
    u_ref[0] = u[:, :hh]
    u_ref[1] = u[:, hh:]


def _k2(h1, s3, cnt3, b1, W2, n, bm):
    grid = (n // bm,)
    hdim = h1.shape[1]
    return pl.pallas_call(
        _k2_body,
        grid=grid,
        in_specs=[
            pl.BlockSpec((bm, hdim), lambda m: (m, 0)),
            pl.BlockSpec((_NC, bm, hdim // 2), lambda m: (0, m, 0)),
            pl.BlockSpec((_NC, bm, 128), lambda m: (0, m, 0)),
            pl.BlockSpec((1, hdim), lambda m: (0, 0)),
            pl.BlockSpec((hdim, hdim), lambda m: (0, 0)),
        ],
        out_specs=[
            pl.BlockSpec((bm, hdim), lambda m: (m, 0)),
            pl.BlockSpec((_NC, bm, hdim // 2), lambda m: (0, m, 0)),
        ],
        out_shape=[
            jax.ShapeDtypeStruct((n, hdim), F32),
            jax.ShapeDtypeStruct((_NC, n, hdim // 2), F32),
        ],
    )(h1, s3, cnt3, b1, W2)


def _k3_body(h_ref, s_ref, cnt_ref, b_ref, wc_ref, bc_ref, z_ref, y_ref):
    sfull = jnp.concatenate([s_ref[0], s_ref[1]], axis=1)
    invs, invd = _deg_terms(cnt_ref[...])
    z = invs * sfull + invd * h_ref[...] + b_ref[...]
    z_ref[...] = z
    y_ref[...] = jnp.dot(z, wc_ref[...], preferred_element_type=F32) + bc_ref[...]


def _k3(h2, s3, cnt3, b2, Wc, bc, n, bm):
    grid = (n // bm,)
    hdim = h2.shape[1]
    return pl.pallas_call(
        _k3_body,
        grid=grid,
        in_specs=[
            pl.BlockSpec((bm, hdim), lambda m: (m, 0)),
            pl.BlockSpec((_NC, bm, hdim // 2), lambda m: (0, m, 0)),
            pl.BlockSpec((_NC, bm, 128), lambda m: (0, m, 0)),
            pl.BlockSpec((1, hdim), lambda m: (0, 0)),
            pl.BlockSpec((hdim, 1), lambda m: (0, 0)),
            pl.BlockSpec((1, 1), lambda m: (0, 0)),
        ],
        out_specs=[
            pl.BlockSpec((bm, hdim), lambda m: (m, 0)),
            pl.BlockSpec((bm, 1), lambda m: (m, 0)),
        ],
        out_shape=[
            jax.ShapeDtypeStruct((n, hdim), F32),
            jax.ShapeDtypeStruct((n, 1), F32),
        ],
    )(h2, s3, cnt3, b2, Wc, bc)


def kernel(x, edge_index, W1, b1, W2, b2, Wc, bc):
    n, d = x.shape
    hdim = W1.shape[1]
    e = edge_index.shape[1]
    half = hdim // 2
    bm = 2000

    src = edge_index[0]
    dst = edge_index[1]
    dst3_deg = dst.reshape(_NC * _NS, e // (_NC * _NS * 40), 40)
    ones = jnp.ones((40, half), F32)
    zeros128 = jnp.zeros((n, half), F32)

    ept = e // _NS
    pad = (-ept) % 240
    src_r = jnp.pad(src.reshape(_NS, ept), ((0, 0), (0, pad)))
    srcs3 = jnp.stack([src_r, src_r + n]).reshape(_NC * _NS, (ept + pad) // 80, 80)
    dst3 = jnp.pad(dst.reshape(_NS, ept), ((0, 0), (0, pad)),
                   constant_values=n).reshape(_NS, (ept + pad) // 80, 80)

    cnt = _deg_call(dst3_deg, ones, zeros128, n, e)
    cnt3 = cnt.reshape(_NC, n, half)

    h1 = _k1a(x, W1, n, bm)
    u1 = _k1b(h1, cnt3, n, bm)
    s1 = _edge_call(u1.reshape(_NC * n, half), srcs3, dst3, zeros128, n, e)
    h2, u2 = _k2(h1, s1.reshape(_NC, n, half), cnt3, b1.reshape(1, hdim), W2, n, bm)
    s2 = _edge_call(u2.reshape(_NC * n, half), srcs3, dst3, zeros128, n, e)
    z, y = _k3(h2, s2.reshape(_NC, n, half), cnt3, b2.reshape(1, hdim),
               Wc, bc.reshape(1, 1), n, bm)
    return (z, y)


# speedup vs baseline: 13.7224x; 1.1740x over previous
import functools

import jax
import jax.numpy as jnp
from jax import lax
from jax.experimental import pallas as pl
from jax.experimental.pallas import tpu as pltpu
from jax.experimental.pallas import tpu_sc as plsc

F32 = jnp.float32

_NC = 2
_NS = 16
_LANES = 16


def _deg_call(dst3, ones, zeros16, n, e):
    w = ones.shape[1]
    chunk = dst3.shape[2]
    nch = dst3.shape[1]
    zrows = n // 10

    def body(dst_hbm, ones_hbm, zeros_hbm, out_hbm, idx_d, ones_v, acc_sh, sem):
        c = lax.axis_index("c")
        s = lax.axis_index("s")

        @pl.when(s < 10)
        def _zero():
            pltpu.sync_copy(zeros_hbm.at[pl.ds(s * zrows, zrows)],
                            acc_sh.at[pl.ds(s * zrows, zrows)])

        pltpu.sync_copy(ones_hbm, ones_v)
        pltpu.sync_copy(dst_hbm.at[c * _NS + s], idx_d)
        plsc.subcore_barrier()

        def fire(i, carry):
            pltpu.async_copy(ones_v, acc_sh.at[idx_d.at[i]], sem, add=True)
            return carry

        lax.fori_loop(0, nch, fire, 0)

        def drain(i, carry):
            pltpu.make_async_copy(ones_v, acc_sh.at[idx_d.at[i]], sem).wait()
            return carry

        lax.fori_loop(0, nch, drain, 0)
        plsc.subcore_barrier()

        @pl.when(s < 10)
        def _writeback():
            pltpu.sync_copy(acc_sh.at[pl.ds(s * zrows, zrows)],
                            out_hbm.at[pl.ds(c * n + s * zrows, zrows)])

    kfn = pl.kernel(
        body,
        out_type=jax.ShapeDtypeStruct((_NC * n, w), F32),
        mesh=plsc.VectorSubcoreMesh(core_axis_name="c", subcore_axis_name="s"),
        scratch_types=[
            pltpu.VMEM((nch, chunk), jnp.int32),
            pltpu.VMEM((chunk, w), F32),
            pltpu.VMEM_SHARED((n, w), F32),
            pltpu.SemaphoreType.DMA,
        ],
    )
    return kfn(dst3, ones, zeros16)


def _edge_call(u2, srcs3, dst3, zeros128, n, e):
    chunk = dst3.shape[2]
    nch = dst3.shape[1]
    zrows = n // 10
    half = u2.shape[1]
    ngrp = nch // 3

    def body(u_hbm, src_hbm, dst_hbm, zeros_hbm, out_hbm,
             is0, is1, is2, idst, rows0, rows1, rows2, acc_sh,
             ise0, ise1, ise2, gse0, gse1, gse2, sse0, sse1, sse2):
        c = lax.axis_index("c")
        s = lax.axis_index("s")

        @pl.when(s < 10)
        def _zero():
            pltpu.sync_copy(zeros_hbm.at[pl.ds(s * zrows, zrows)],
                            acc_sh.at[pl.ds(s * zrows, zrows)])

        t = c * _NS + s
        pltpu.sync_copy(dst_hbm.at[s], idst)
        plsc.subcore_barrier()

        isb = (is0, is1, is2)
        rows = (rows0, rows1, rows2)
        isem = (ise0, ise1, ise2)
        gsem = (gse0, gse1, gse2)
        ssem = (sse0, sse1, sse2)

        def idxload(i, b):
            pltpu.async_copy(src_hbm.at[t].at[i], isb[b], isem[b])

        def wait_idx(i, b):
            pltpu.make_async_copy(src_hbm.at[t].at[i], isb[b], isem[b]).wait()

        def gather(i, b):
            pltpu.async_copy(u_hbm.at[isb[b]], rows[b], gsem[b])

        def wait_gather(i, b):
            pltpu.make_async_copy(u_hbm.at[isb[b]], rows[b], gsem[b]).wait()

        def scatter(i, b):
            pltpu.async_copy(rows[b], acc_sh.at[idst.at[i]], ssem[b], add=True)

        def wait_scatter(i, b):
            pltpu.make_async_copy(rows[b], acc_sh.at[idst.at[i]], ssem[b]).wait()

        idxload(0, 0)
        idxload(1, 1)
        wait_idx(0, 0)
        gather(0, 0)
        idxload(2, 2)
        wait_idx(1, 1)
        gather(1, 1)
        wait_gather(0, 0)
        scatter(0, 0)
        idxload(3, 0)
        wait_idx(2, 2)
        gather(2, 2)
        wait_gather(1, 1)
        scatter(1, 1)
        idxload(4, 1)

        def grp(g, carry):
            for b in range(3):
                i = 3 * g + b
                bp = (b + 2) % 3
                wait_idx(i, b)
                wait_scatter(i - 3, b)
                gather(i, b)
                wait_gather(i - 1, bp)
                scatter(i - 1, bp)
                if b == 0:
                    idxload(i + 2, bp)
                else:
                    @pl.when(g < ngrp - 1)
                    def _pf():
                        idxload(i + 2, bp)
            return carry

        lax.fori_loop(1, ngrp, grp, 0)
        wait_gather(nch - 1, (nch - 1) % 3)
        scatter(nch - 1, (nch - 1) % 3)
        wait_scatter(nch - 3, (nch - 3) % 3)
        wait_scatter(nch - 2, (nch - 2) % 3)
        wait_scatter(nch - 1, (nch - 1) % 3)
        plsc.subcore_barrier()

        @pl.when(s < 10)
        def _writeback():
            pltpu.sync_copy(acc_sh.at[pl.ds(s * zrows, zrows)],
                            out_hbm.at[pl.ds(c * n + s * zrows, zrows)])

    kfn = pl.kernel(
        body,
        out_type=jax.ShapeDtypeStruct((_NC * n, half), F32),
        mesh=plsc.VectorSubcoreMesh(core_axis_name="c", subcore_axis_name="s"),
        scratch_types=[
            pltpu.VMEM((chunk,), jnp.int32),
            pltpu.VMEM((chunk,), jnp.int32),
            pltpu.VMEM((chunk,), jnp.int32),
            pltpu.VMEM((nch, chunk), jnp.int32),
            pltpu.VMEM((chunk, half), F32),
            pltpu.VMEM((chunk, half), F32),
            pltpu.VMEM((chunk, half), F32),
            pltpu.VMEM_SHARED((n + 8, half), F32),
            pltpu.SemaphoreType.DMA,
            pltpu.SemaphoreType.DMA,
            pltpu.SemaphoreType.DMA,
            pltpu.SemaphoreType.DMA,
            pltpu.SemaphoreType.DMA,
            pltpu.SemaphoreType.DMA,
            pltpu.SemaphoreType.DMA,
            pltpu.SemaphoreType.DMA,
            pltpu.SemaphoreType.DMA,
        ],
    )
    return kfn(u2, srcs3, dst3, zeros128)


def _deg_terms(cnt_blk):
    deg = 1.0 + cnt_blk[0, :, 0:1] + cnt_blk[1, :, 0:1]
    return lax.rsqrt(deg), 1.0 / deg


def _k1a_body(x_ref, w_ref, h_ref):
    h_ref[...] = jnp.dot(x_ref[...], w_ref[...], preferred_element_type=F32)


def _k1a(x, W1, n, bm):
    grid = (n // bm,)
    d = x.shape[1]
    hdim = W1.shape[1]
    return pl.pallas_call(
        _k1a_body,
        grid=grid,
        in_specs=[
            pl.BlockSpec((bm, d), lambda m: (m, 0)),
            pl.BlockSpec((d, hdim), lambda m: (0, 0)),
        ],
        out_specs=pl.BlockSpec((bm, hdim), lambda m: (m, 0)),
        out_shape=jax.ShapeDtypeStruct((n, hdim), F32),
    )(x, W1)


def _k1b_body(h_ref, cnt_ref, u_ref):
    invs, _ = _deg_terms(cnt_ref[...])
    u = invs * h_ref[...]
    hh = u.shape[1] // 2
    u_ref[0] = u[:, :hh]
    u_ref[1] = u[:, hh:]


def _k1b(h1, cnt3, n, bm):
    grid = (n // bm,)
    hdim = h1.shape[1]
    return pl.pallas_call(
        _k1b_body,
        grid=grid,
        in_specs=[
            pl.BlockSpec((bm, hdim), lambda m: (m, 0)),
            pl.BlockSpec((_NC, bm, 128), lambda m: (0, m, 0)),
        ],
        out_specs=pl.BlockSpec((_NC, bm, hdim // 2), lambda m: (0, m, 0)),
        out_shape=jax.ShapeDtypeStruct((_NC, n, hdim // 2), F32),
    )(h1, cnt3)


def _k2_body(h_ref, s_ref, cnt_ref, b_ref, w_ref, h2_ref, u_ref):
    sfull = jnp.concatenate([s_ref[0], s_ref[1]], axis=1)
    invs, invd = _deg_terms(cnt_ref[...])
    pre = invs * sfull + invd * h_ref[...] + b_ref[...]
    a = jnp.maximum(pre, 0.0)
    h2 = jnp.dot(a, w_ref[...], preferred_element_type=F32)
    h2_ref[...] = h2
    u = invs * h2
    hh = u.shape[1] // 2
    u_ref[0] = u[:, :hh]
    u_ref[1] = u[:, hh:]


def _k2(h1, s3, cnt3, b1, W2, n, bm):
    grid = (n // bm,)
    hdim = h1.shape[1]
    return pl.pallas_call(
        _k2_body,
        grid=grid,
        in_specs=[
            pl.BlockSpec((bm, hdim), lambda m: (m, 0)),
            pl.BlockSpec((_NC, bm, hdim // 2), lambda m: (0, m, 0)),
            pl.BlockSpec((_NC, bm, 128), lambda m: (0, m, 0)),
            pl.BlockSpec((1, hdim), lambda m: (0, 0)),
            pl.BlockSpec((hdim, hdim), lambda m: (0, 0)),
        ],
        out_specs=[
            pl.BlockSpec((bm, hdim), lambda m: (m, 0)),
            pl.BlockSpec((_NC, bm, hdim // 2), lambda m: (0, m, 0)),
        ],
        out_shape=[
            jax.ShapeDtypeStruct((n, hdim), F32),
            jax.ShapeDtypeStruct((_NC, n, hdim // 2), F32),
        ],
    )(h1, s3, cnt3, b1, W2)


def _k3_body(h_ref, s_ref, cnt_ref, b_ref, wc_ref, bc_ref, z_ref, y_ref):
    sfull = jnp.concatenate([s_ref[0], s_ref[1]], axis=1)
    invs, invd = _deg_terms(cnt_ref[...])
    z = invs * sfull + invd * h_ref[...] + b_ref[...]
    z_ref[...] = z
    y_ref[...] = jnp.dot(z, wc_ref[...], preferred_element_type=F32) + bc_ref[...]


def _k3(h2, s3, cnt3, b2, Wc, bc, n, bm):
    grid = (n // bm,)
    hdim = h2.shape[1]
    return pl.pallas_call(
        _k3_body,
        grid=grid,
        in_specs=[
            pl.BlockSpec((bm, hdim), lambda m: (m, 0)),
            pl.BlockSpec((_NC, bm, hdim // 2), lambda m: (0, m, 0)),
            pl.BlockSpec((_NC, bm, 128), lambda m: (0, m, 0)),
            pl.BlockSpec((1, hdim), lambda m: (0, 0)),
            pl.BlockSpec((hdim, 1), lambda m: (0, 0)),
            pl.BlockSpec((1, 1), lambda m: (0, 0)),
        ],
        out_specs=[
            pl.BlockSpec((bm, hdim), lambda m: (m, 0)),
            pl.BlockSpec((bm, 1), lambda m: (m, 0)),
        ],
        out_shape=[
            jax.ShapeDtypeStruct((n, hdim), F32),
            jax.ShapeDtypeStruct((n, 1), F32),
        ],
    )(h2, s3, cnt3, b2, Wc, bc)


def kernel(x, edge_index, W1, b1, W2, b2, Wc, bc):
    n, d = x.shape
    hdim = W1.shape[1]
    e = edge_index.shape[1]
    half = hdim // 2
    bm = 2000

    src = edge_index[0]
    dst = edge_index[1]
    dst3_deg = dst.reshape(_NC * _NS, e // (_NC * _NS * 40), 40)
    ones = jnp.ones((40, half), F32)
    zeros128 = jnp.zeros((n, half), F32)

    ept = e // _NS
    ck = 88
    pad = (-ept) % (3 * ck)
    src_r = jnp.pad(src.reshape(_NS, ept), ((0, 0), (0, pad)))
    srcs3 = jnp.stack([src_r, src_r + n]).reshape(_NC * _NS, (ept + pad) // ck, ck)
    dst3 = jnp.pad(dst.reshape(_NS, ept), ((0, 0), (0, pad)),
                   constant_values=n).reshape(_NS, (ept + pad) // ck, ck)

    cnt = _deg_call(dst3_deg, ones, zeros128, n, e)
    cnt3 = cnt.reshape(_NC, n, half)

    h1 = _k1a(x, W1, n, bm)
    u1 = _k1b(h1, cnt3, n, bm)
    s1 = _edge_call(u1.reshape(_NC * n, half), srcs3, dst3, zeros128, n, e)
    h2, u2 = _k2(h1, s1.reshape(_NC, n, half), cnt3, b1.reshape(1, hdim), W2, n, bm)
    s2 = _edge_call(u2.reshape(_NC * n, half), srcs3, dst3, zeros128, n, e)
    z, y = _k3(h2, s2.reshape(_NC, n, half), cnt3, b2.reshape(1, hdim),
               Wc, bc.reshape(1, 1), n, bm)
    return (z, y)
